# baseline (speedup 1.0000x reference)
import functools
import math

import jax
import jax.numpy as jnp
from jax import lax
from jax.experimental import pallas as pl
from jax.experimental.pallas import tpu as pltpu
from jax.experimental.pallas import tpu_sc as plsc

D_MODEL = 64
SCALE = math.sqrt(D_MODEL)

_NC = 2
_NS = 16
_NW = _NC * _NS
_NB = 2


@functools.partial(jax.jit, static_argnames=("nrows", "seq"))
def _sc_embed(table2, xf, *, nrows, seq):
    n_idx = nrows * seq
    per_w = n_idx // _NW
    rows_per_w = nrows // _NW
    n = rows_per_w
    mesh = plsc.VectorSubcoreMesh(core_axis_name="c", subcore_axis_name="s")

    @functools.partial(
        pl.kernel,
        mesh=mesh,
        out_type=jax.ShapeDtypeStruct((nrows, seq, D_MODEL), jnp.float32),
        scratch_types=[
            pltpu.VMEM((per_w + 16,), jnp.int32),
        ]
        + [pltpu.VMEM((208,), jnp.int32) for _ in range(_NB)]
        + [pltpu.VMEM((seq, 2 * D_MODEL), jnp.float32) for _ in range(_NB)]
        + [pltpu.VMEM((seq, D_MODEL), jnp.float32) for _ in range(_NB)]
        + [pltpu.SemaphoreType.DMA for _ in range(2 * _NB)],
        compiler_params=pltpu.CompilerParams(use_tc_tiling_on_sc=True),
    )
    def k(table_hbm, idx_hbm, out_hbm, idx_v, *rest):
        idx2 = rest[:_NB]
        in_buf = rest[_NB:2 * _NB]
        out_buf = rest[2 * _NB:3 * _NB]
        sem_g = rest[3 * _NB:4 * _NB]
        sem_s = rest[4 * _NB:]

        wid = lax.axis_index("s") * _NC + lax.axis_index("c")
        wbase = wid * per_w
        row0 = wid * rows_per_w
        pltpu.sync_copy(idx_hbm.at[pl.ds(wbase, per_w)],
                        idx_v.at[pl.ds(0, per_w)])

        def start_gather(g, b):
            def prep(q, c2):
                v = idx_v[pl.ds(g * seq + q * 16, 16)]
                idx2[b][pl.ds(q * 16, 16)] = lax.shift_right_logical(v, 1)
                return c2
            lax.fori_loop(0, (seq + 15) // 16, prep, 0)
            return pltpu.async_copy(
                table_hbm.at[idx2[b].at[pl.ds(0, seq)]],
                in_buf[b], sem_g[b])

        def wait_gather(b):
            pltpu.make_async_copy(
                table_hbm.at[idx2[b].at[pl.ds(0, seq)]],
                in_buf[b], sem_g[b]).wait()

        def start_scatter(g, b):
            return pltpu.async_copy(out_buf[b], out_hbm.at[row0 + g],
                                    sem_s[b])

        def wait_scatter(b):
            pltpu.make_async_copy(out_buf[b], out_hbm.at[row0],
                                  sem_s[b]).wait()

        def compute(g, b):
            def row(j, c2):
                iv = idx_v[pl.ds(g * seq + j, 16)]
                sel64 = (iv[0] & 1) * D_MODEL
                for q in range(D_MODEL // 16):
                    src = in_buf[b][j, pl.ds(sel64 + q * 16, 16)]
                    out_buf[b][j, pl.ds(q * 16, 16)] = src * SCALE
                return c2
            lax.fori_loop(0, seq, row, 0, unroll=2)

        start_gather(0, 0)
        start_gather(1, 1)
        wait_gather(0)
        compute(0, 0)
        start_scatter(0, 0)
        start_gather(2, 0)
        wait_gather(1)
        compute(1, 1)
        start_scatter(1, 1)

        def steady(p, carry):
            for t in range(_NB):
                g = 2 + p * _NB + t
                b = t
                start_gather(g + 1, 1 - t)
                wait_gather(b)
                wait_scatter(b)
                compute(g, b)
                start_scatter(g, b)
            return carry

        lax.fori_loop(0, (n - 4) // _NB, steady, 0)

        for g in (n - 2, n - 1):
            b = g % _NB
            if g + 1 < n:
                start_gather(g + 1, (g + 1) % _NB)
            wait_gather(b)
            wait_scatter(b)
            compute(g, b)
            start_scatter(g, b)
        for b in range(_NB):
            wait_scatter(b)

    return k(table2, xf)


def kernel(x, table):
    if x.dtype != jnp.int32:
        x = x.astype(jnp.int32)
    table2 = table.reshape(table.shape[0] // 2, 2 * table.shape[1])
    xf = x.reshape(x.shape[0] * x.shape[1])
    return _sc_embed(table2, xf, nrows=x.shape[0], seq=x.shape[1])
